# baseline (speedup 1.0000x reference)
import functools

import jax
import jax.numpy as jnp
from jax import lax
from jax.experimental import pallas as pl
from jax.experimental.pallas import tpu as pltpu
from jax.experimental.pallas import tpu_sc as plsc

N_TOKENS = 32768
DIM = 2048
N_EXPERTS = 8
ROUTE_SCALE = 1.0

BLK = 1024

NC = 2
NS = 16
NW = NC * NS
TPW = N_TOKENS // NW
GROUPS = TPW // 16


def _scores_body(x_ref, wt_ref, s_ref):
    s_ref[...] = jnp.dot(x_ref[...], wt_ref[...],
                         preferred_element_type=jnp.float32)


def _tc_scores(x, wt):
    return pl.pallas_call(
        _scores_body,
        grid=(N_TOKENS // BLK,),
        in_specs=[
            pl.BlockSpec((BLK, DIM), lambda i: (i, 0)),
            pl.BlockSpec((DIM, N_EXPERTS), lambda i: (0, 0)),
        ],
        out_specs=pl.BlockSpec((BLK, N_EXPERTS), lambda i: (i, 0)),
        out_shape=jax.ShapeDtypeStruct((N_TOKENS, N_EXPERTS), jnp.float32),
    )(x, wt)


@functools.partial(
    pl.kernel,
    mesh=plsc.VectorSubcoreMesh(core_axis_name="c", subcore_axis_name="s"),
    out_type=[
        jax.ShapeDtypeStruct((N_TOKENS * 2,), jnp.float32),
        jax.ShapeDtypeStruct((N_TOKENS * 2,), jnp.int32),
    ],
    scratch_types=[
        pltpu.VMEM((TPW * N_EXPERTS,), jnp.float32),
        pltpu.VMEM((TPW * 2,), jnp.float32),
        pltpu.VMEM((TPW * 2,), jnp.int32),
    ],
    compiler_params=pltpu.CompilerParams(needs_layout_passes=False),
)
def _sc_router(scores_hbm, w_hbm, i_hbm, s_v, w_v, i_v):
    wid = lax.axis_index("s") * NC + lax.axis_index("c")
    pltpu.sync_copy(scores_hbm.at[pl.ds(wid * (TPW * N_EXPERTS),
                                        TPW * N_EXPERTS)], s_v)

    def group(g, _):
        tok = g * 16 + lax.broadcasted_iota(jnp.int32, (16,), 0)
        s = [plsc.load_gather(s_v, [tok * N_EXPERTS + e])
             for e in range(N_EXPERTS)]
        m1 = s[0]
        i1 = jnp.zeros((16,), jnp.int32)
        m2 = jnp.full((16,), -jnp.inf, jnp.float32)
        i2 = jnp.zeros((16,), jnp.int32)
        for e in range(1, N_EXPERTS):
            se = s[e]
            ev = jnp.full((16,), e, jnp.int32)
            gt1 = se > m1
            gt2 = se > m2
            m2 = jnp.where(gt1, m1, jnp.where(gt2, se, m2))
            i2 = jnp.where(gt1, i1, jnp.where(gt2, ev, i2))
            m1 = jnp.where(gt1, se, m1)
            i1 = jnp.where(gt1, ev, i1)
        denom = jnp.zeros((16,), jnp.float32)
        for e in range(N_EXPERTS):
            denom = denom + jnp.exp(s[e] - m1)
        w1 = (1.0 / denom) * ROUTE_SCALE
        w2 = (jnp.exp(m2 - m1) / denom) * ROUTE_SCALE
        plsc.store_scatter(w_v, [tok * 2], w1)
        plsc.store_scatter(w_v, [tok * 2 + 1], w2)
        plsc.store_scatter(i_v, [tok * 2], i1)
        plsc.store_scatter(i_v, [tok * 2 + 1], i2)
        return 0

    lax.fori_loop(0, GROUPS, group, 0)
    pltpu.sync_copy(w_v, w_hbm.at[pl.ds(wid * (TPW * 2), TPW * 2)])
    pltpu.sync_copy(i_v, i_hbm.at[pl.ds(wid * (TPW * 2), TPW * 2)])


PROBE_CHUNK = 32768
PROBE_CHUNKS = (N_TOKENS * DIM) // NW // PROBE_CHUNK


@functools.partial(
    pl.kernel,
    mesh=plsc.VectorSubcoreMesh(core_axis_name="c", subcore_axis_name="s"),
    out_type=jax.ShapeDtypeStruct((NW * 8,), jnp.float32),
    scratch_types=[
        pltpu.VMEM((PROBE_CHUNK,), jnp.float32),
        pltpu.VMEM((PROBE_CHUNK,), jnp.float32),
        pltpu.VMEM((PROBE_CHUNK,), jnp.float32),
        pltpu.SemaphoreType.DMA,
        pltpu.SemaphoreType.DMA,
        pltpu.SemaphoreType.DMA,
    ],
    compiler_params=pltpu.CompilerParams(needs_layout_passes=False),
)
def _sc_stream_probe(x_hbm, out_hbm, s0, s1, s2, m0, m1, m2):
    wid = lax.axis_index("s") * NC + lax.axis_index("c")
    base = wid * (PROBE_CHUNKS * PROBE_CHUNK)
    bufs = [s0, s1, s2]
    sems = [m0, m1, m2]
    copies = []
    for c in range(PROBE_CHUNKS):
        if c >= 3:
            copies[c - 3].wait()
        copies.append(pltpu.async_copy(
            x_hbm.at[pl.ds(base + c * PROBE_CHUNK, PROBE_CHUNK)],
            bufs[c % 3], sems[c % 3]))
    for c in range(PROBE_CHUNKS - 3, PROBE_CHUNKS):
        copies[c].wait()
    pltpu.sync_copy(s0.at[pl.ds(0, 8)], out_hbm.at[pl.ds(wid * 8, 8)])


def kernel(x, W):
    dummy = _sc_stream_probe(x.reshape(-1))
    scores = _tc_scores(x, W.T)
    w_flat, i_flat = _sc_router(scores.reshape(-1))
    weights = w_flat.reshape(N_TOKENS, 2).astype(x.dtype)
    weights = weights + dummy[0] * 0.0
    indices = i_flat.reshape(N_TOKENS, 2)
    return weights, indices


# speedup vs baseline: 1.3880x; 1.3880x over previous
import functools

import jax
import jax.numpy as jnp
from jax import lax
from jax.experimental import pallas as pl
from jax.experimental.pallas import tpu as pltpu
from jax.experimental.pallas import tpu_sc as plsc

N_TOKENS = 32768
DIM = 2048
N_EXPERTS = 8
ROUTE_SCALE = 1.0

NC = 2
NS = 16
NW = NC * NS

N_SC = 13312
N_TC = N_TOKENS - N_SC

BLK = 1024

TPW = N_TC // NW
GROUPS = TPW // 16

SC_TPW = N_SC // NW
SC_CH = 16
SC_NCH = SC_TPW // SC_CH
DCH = DIM // 16


def _scores_body(x_ref, wt_ref, s_ref):
    s_ref[...] = jnp.dot(x_ref[...], wt_ref[...],
                         preferred_element_type=jnp.float32)


def _tc_scores(x, wt):
    return pl.pallas_call(
        _scores_body,
        grid=(N_TC // BLK,),
        in_specs=[
            pl.BlockSpec((BLK, DIM), lambda i: (i, 0)),
            pl.BlockSpec((DIM, N_EXPERTS), lambda i: (0, 0)),
        ],
        out_specs=pl.BlockSpec((BLK, N_EXPERTS), lambda i: (i, 0)),
        out_shape=jax.ShapeDtypeStruct((N_TC, N_EXPERTS), jnp.float32),
    )(x, wt)


def _lane_iota():
    return lax.broadcasted_iota(jnp.int32, (16,), 0)


def _bf16_rne(v):
    u = plsc.bitcast(v, jnp.uint32)
    tie = lax.shift_right_logical(u, jnp.uint32(16)) & jnp.uint32(1)
    u = (u + jnp.uint32(0x7FFF) + tie) & jnp.uint32(0xFFFF0000)
    return plsc.bitcast(u, jnp.float32)


def _route16(s, out_w, out_i, tok):
    m1 = s[0]
    i1 = jnp.zeros((16,), jnp.int32)
    m2 = jnp.full((16,), -jnp.inf, jnp.float32)
    i2 = jnp.zeros((16,), jnp.int32)
    for e in range(1, N_EXPERTS):
        se = s[e]
        ev = jnp.full((16,), e, jnp.int32)
        gt1 = se > m1
        gt2 = se > m2
        m2 = jnp.where(gt1, m1, jnp.where(gt2, se, m2))
        i2 = jnp.where(gt1, i1, jnp.where(gt2, ev, i2))
        m1 = jnp.where(gt1, se, m1)
        i1 = jnp.where(gt1, ev, i1)
    denom = jnp.zeros((16,), jnp.float32)
    for e in range(N_EXPERTS):
        denom = denom + jnp.exp(s[e] - m1)
    w1 = (1.0 / denom) * ROUTE_SCALE
    w2 = (jnp.exp(m2 - m1) / denom) * ROUTE_SCALE
    plsc.store_scatter(out_w, [tok * 2], w1)
    plsc.store_scatter(out_w, [tok * 2 + 1], w2)
    plsc.store_scatter(out_i, [tok * 2], i1)
    plsc.store_scatter(out_i, [tok * 2 + 1], i2)


@functools.partial(
    pl.kernel,
    mesh=plsc.VectorSubcoreMesh(core_axis_name="c", subcore_axis_name="s"),
    out_type=[
        jax.ShapeDtypeStruct((N_TC * 2,), jnp.float32),
        jax.ShapeDtypeStruct((N_TC * 2,), jnp.int32),
    ],
    scratch_types=[
        pltpu.VMEM((TPW * N_EXPERTS,), jnp.float32),
        pltpu.VMEM((TPW * 2,), jnp.float32),
        pltpu.VMEM((TPW * 2,), jnp.int32),
    ],
    compiler_params=pltpu.CompilerParams(needs_layout_passes=False),
)
def _sc_router(scores_hbm, w_hbm, i_hbm, s_v, w_v, i_v):
    wid = lax.axis_index("s") * NC + lax.axis_index("c")
    pltpu.sync_copy(scores_hbm.at[pl.ds(wid * (TPW * N_EXPERTS),
                                        TPW * N_EXPERTS)], s_v)

    def group(g, _):
        tok = g * 16 + _lane_iota()
        s = [plsc.load_gather(s_v, [tok * N_EXPERTS + e])
             for e in range(N_EXPERTS)]
        _route16(s, w_v, i_v, tok)
        return 0

    lax.fori_loop(0, GROUPS, group, 0)
    pltpu.sync_copy(w_v, w_hbm.at[pl.ds(wid * (TPW * 2), TPW * 2)])
    pltpu.sync_copy(i_v, i_hbm.at[pl.ds(wid * (TPW * 2), TPW * 2)])


@functools.partial(
    pl.kernel,
    mesh=plsc.VectorSubcoreMesh(core_axis_name="c", subcore_axis_name="s"),
    out_type=[
        jax.ShapeDtypeStruct((N_SC * 2,), jnp.float32),
        jax.ShapeDtypeStruct((N_SC * 2,), jnp.int32),
    ],
    scratch_types=[
        pltpu.VMEM((N_EXPERTS, DIM), jnp.float32),
        pltpu.VMEM((SC_CH, DIM), jnp.float32),
        pltpu.VMEM((SC_CH, DIM), jnp.float32),
        pltpu.VMEM((SC_CH * N_EXPERTS,), jnp.float32),
        pltpu.VMEM((SC_TPW * 2,), jnp.float32),
        pltpu.VMEM((SC_TPW * 2,), jnp.int32),
        pltpu.SemaphoreType.DMA,
        pltpu.SemaphoreType.DMA,
    ],
    compiler_params=pltpu.CompilerParams(needs_layout_passes=False),
)
def _sc_matmul_router(x_hbm, w_full_hbm, wout_hbm, iout_hbm,
                      wv, xb0, xb1, s_out, w_v, i_v, sem0, sem1):
    wid = lax.axis_index("s") * NC + lax.axis_index("c")
    row0 = N_TC + wid * SC_TPW
    bufs = (xb0, xb1)
    sems = (sem0, sem1)
    lane = _lane_iota()

    pltpu.sync_copy(w_full_hbm, wv)
    pltpu.async_copy(x_hbm.at[pl.ds(row0, SC_CH)], xb0, sem0)

    def wtrunc(dd, _):
        d = dd * 16
        for e in range(N_EXPERTS):
            wv[e, pl.ds(d, 16)] = _bf16_rne(wv[e, pl.ds(d, 16)])
        return 0

    lax.fori_loop(0, DCH, wtrunc, 0)

    def chunk_pair(cc, _):
        for k in range(2):
            c = cc * 2 + k
            buf = bufs[k]
            pltpu.make_async_copy(
                x_hbm.at[pl.ds(row0, SC_CH)], buf, sems[k]).wait()
            @pl.when(c + 1 < SC_NCH)
            def _():
                pltpu.async_copy(
                    x_hbm.at[pl.ds(row0 + (c + 1) * SC_CH, SC_CH)],
                    bufs[1 - k], sems[1 - k])
            for tg in range(SC_CH // 4):
                def dbody(dd, accs):
                    d = dd * 16
                    xs = [_bf16_rne(buf[tg * 4 + i, pl.ds(d, 16)])
                          for i in range(4)]
                    ws = [wv[e, pl.ds(d, 16)] for e in range(N_EXPERTS)]
                    return tuple(
                        accs[i * N_EXPERTS + e] + xs[i] * ws[e]
                        for i in range(4) for e in range(N_EXPERTS))

                accs = lax.fori_loop(
                    0, DCH, dbody,
                    tuple(jnp.zeros((16,), jnp.float32) for _ in range(32)))
                for p in range(2):
                    pair = jnp.zeros((16,), jnp.float32)
                    for i in range(2):
                        for e in range(N_EXPERTS):
                            tot = jnp.sum(accs[(p * 2 + i) * N_EXPERTS + e])
                            pair = jnp.where(lane == i * 8 + e, tot, pair)
                    s_out[pl.ds((tg * 4 + p * 2) * N_EXPERTS, 16)] = pair
            tok_l = lane
            s = [plsc.load_gather(s_out, [tok_l * N_EXPERTS + e])
                 for e in range(N_EXPERTS)]
            _route16(s, w_v, i_v, c * SC_CH + tok_l)
        return 0

    lax.fori_loop(0, SC_NCH // 2, chunk_pair, 0)
    pltpu.sync_copy(w_v, wout_hbm.at[pl.ds(wid * (SC_TPW * 2), SC_TPW * 2)])
    pltpu.sync_copy(i_v, iout_hbm.at[pl.ds(wid * (SC_TPW * 2), SC_TPW * 2)])


def kernel(x, W):
    w_sc, i_sc = _sc_matmul_router(x, W)
    scores_tc = _tc_scores(x, W.T)
    w_tc, i_tc = _sc_router(scores_tc.reshape(-1))
    weights = jnp.concatenate([w_tc, w_sc]).reshape(N_TOKENS, 2)
    indices = jnp.concatenate([i_tc, i_sc]).reshape(N_TOKENS, 2)
    return weights.astype(x.dtype), indices
